# Initial kernel scaffold; baseline (speedup 1.0000x reference)
import jax
import jax.numpy as jnp
from jax.experimental import pallas as pl


def kernel(pre_x, pre_edge_index, pre_edge_attr, pre_batch, post_x, post_edge_index, post_edge_attr, post_batch, shots, backend_feat, circuit_feat, graph_attr_feat, params):
    raise NotImplementedError("write your pallas kernel here")


# speedup vs baseline: 1.3300x; 1.3300x over previous
import functools

import jax
import jax.numpy as jnp
from jax import lax
from jax.experimental import pallas as pl
from jax.experimental.pallas import tpu as pltpu

HID = 64
N_NODES = 10000
N_EDGES = 320000
N_GRAPHS = 64
N_LAYERS = 4

_MLP_BLK = 2000


def _mlp_body(x_ref, a_ref, w0_ref, b0_ref, w1_ref, b1_ref, sc_ref, bt_ref, o_ref):
    x = x_ref[...]
    y = x + a_ref[...]
    h = jnp.dot(y, w0_ref[...], preferred_element_type=jnp.float32) + b0_ref[...]
    h = jnp.maximum(h, 0.0)
    h = jnp.dot(h, w1_ref[...], preferred_element_type=jnp.float32) + b1_ref[...]
    h = h * sc_ref[...] + bt_ref[...]
    o_ref[...] = x + jnp.maximum(h, 0.0)


def _mlp_layer(x, aggr, conv, norm):
    scale = (norm["gamma"] / jnp.sqrt(1.0 + 1e-5)).reshape(1, HID)
    beta = norm["beta"].reshape(1, HID)
    b0 = conv["l0"]["b"].reshape(1, HID)
    b1 = conv["l1"]["b"].reshape(1, HID)
    nblk = N_NODES // _MLP_BLK
    row_spec = pl.BlockSpec((_MLP_BLK, HID), lambda i: (i, 0))
    full = lambda shape: pl.BlockSpec(shape, lambda i: (0, 0))
    return pl.pallas_call(
        _mlp_body,
        grid=(nblk,),
        in_specs=[row_spec, row_spec, full((HID, HID)), full((1, HID)),
                  full((HID, HID)), full((1, HID)), full((1, HID)), full((1, HID))],
        out_specs=row_spec,
        out_shape=jax.ShapeDtypeStruct((N_NODES, HID), jnp.float32),
    )(x, aggr, conv["l0"]["w"], b0, conv["l1"]["w"], b1, scale, beta)


def _gine_aggr(x, ea, src, dst):
    msg = jax.nn.relu(x[src] + ea)
    return jax.ops.segment_sum(msg, dst, num_segments=N_NODES)


def _encoder(p, x, edge_index, edge_attr, batch):
    x = x @ p["node_proj"]["w"] + p["node_proj"]["b"]
    ea = edge_attr @ p["edge_proj"]["w"] + p["edge_proj"]["b"]
    src = edge_index[0]
    dst = edge_index[1]
    for i in range(N_LAYERS):
        aggr = _gine_aggr(x, ea, src, dst)
        x = _mlp_layer(x, aggr, p["convs"][i], p["norms"][i])
    ssum = jax.ops.segment_sum(x, batch, num_segments=N_GRAPHS)
    cnt = jax.ops.segment_sum(jnp.ones((N_NODES,), jnp.float32), batch, num_segments=N_GRAPHS)
    g_mean = ssum / jnp.maximum(cnt, 1.0)[:, None]
    g_max = jax.ops.segment_max(x, batch, num_segments=N_GRAPHS)
    g_max = jnp.where(cnt[:, None] > 0, g_max, 0.0)
    return jnp.concatenate([g_mean, g_max], axis=-1)


def _linear(p, x):
    return x @ p["w"] + p["b"]


def _mlp2(p, x):
    return _linear(p["l1"], jax.nn.relu(_linear(p["l0"], x)))


def _feat_enc(p, x):
    return jax.nn.relu(_linear(p["l1"], jax.nn.relu(_linear(p["l0"], x))))


def kernel(pre_x, pre_edge_index, pre_edge_attr, pre_batch, post_x,
           post_edge_index, post_edge_attr, post_batch, shots, backend_feat,
           circuit_feat, graph_attr_feat, params):
    g_pre = _encoder(params["pre_enc"], pre_x, pre_edge_index, pre_edge_attr, pre_batch)
    g_post = _encoder(params["post_enc"], post_x, post_edge_index, post_edge_attr, post_batch)
    pair = jnp.concatenate([g_pre, g_post], axis=-1)
    gate_pre = jax.nn.sigmoid(_mlp2(params["fusion_pre"], pair))
    gate_post = jax.nn.sigmoid(_mlp2(params["fusion_post"], pair))
    gp = gate_pre * g_pre
    gq = gate_post * g_post
    g_pair = jnp.concatenate([gp, gq, jnp.abs(gp - gq), gp * gq], axis=-1)
    bz = _feat_enc(params["backend_enc"], backend_feat)
    cz = _feat_enc(params["circuit_enc"], circuit_feat)
    gz = _feat_enc(params["gattr_enc"], graph_attr_feat)
    h = jnp.concatenate([g_pair, bz, cz, gz], axis=-1)
    h = jax.nn.relu(_linear(params["backbone"]["l0"], h))
    h = jax.nn.relu(_linear(params["backbone"]["l1"], h))
    f_inf = jax.nn.sigmoid(_linear(params["f_inf"], h))
    theta = jax.nn.softplus(_linear(params["theta"], h)) + 1.0
    log_c = _linear(params["log_c"], h)
    s = shots.reshape(-1, 1).astype(jnp.float32)
    g = s / (s + theta)
    mean = (f_inf * g).squeeze(-1)
    logvar = (log_c - jnp.log(s)).squeeze(-1)
    return (mean, logvar)


# speedup vs baseline: 3.0092x; 2.2626x over previous
import jax
import jax.numpy as jnp
from jax import lax
from jax.experimental import pallas as pl
from jax.experimental.pallas import tpu as pltpu
from jax.experimental.pallas import tpu_sc as plsc

HID = 64
N_NODES = 10000
N_EDGES = 320000
N_GRAPHS = 64
N_LAYERS = 4

_MLP_BLK = 2000

_C = 128
_NCHUNK = N_EDGES // _C
_NW = 32
_CBASE = _NCHUNK // _NW
_CEXTRA = _NCHUNK % _NW
_RPT = 632
_RPT_LAST = N_NODES - 15 * _RPT


def _sc_layer_body(xp_hbm, xq_hbm, eap_hbm, eaq_hbm, sp_hbm, dp_hbm, sq_hbm,
                   dq_hbm, out_hbm, ap_sh, aq_sh, sidx_v, didx_v, xg, eab,
                   zbuf, gsems, easems, ssems, dsems):
    cid = lax.axis_index("c")
    sid = lax.axis_index("s")
    wid = sid * 2 + cid
    c0 = wid * _CBASE + jnp.minimum(wid, _CEXTRA)
    nc = _CBASE + jnp.where(wid < _CEXTRA, 1, 0)
    r0 = sid * _RPT
    nrows = jnp.where(sid < 15, _RPT, _RPT_LAST)

    zero16 = jnp.zeros((16,), jnp.float32)

    def _zrow(r, _):
        for k in range(4):
            zbuf[r, pl.ds(k * 16, 16)] = zero16
        return 0

    lax.fori_loop(0, 8, _zrow, 0)

    def _zcopy(n, _):
        pltpu.sync_copy(zbuf, ap_sh.at[pl.ds(r0 + n * 8, 8), :])
        pltpu.sync_copy(zbuf, aq_sh.at[pl.ds(r0 + n * 8, 8), :])
        return 0

    lax.fori_loop(0, nrows // 8, _zcopy, 0)
    plsc.subcore_barrier()

    for x_hbm, ea_hbm, s_hbm, d_hbm, a_sh in (
            (xp_hbm, eap_hbm, sp_hbm, dp_hbm, ap_sh),
            (xq_hbm, eaq_hbm, sq_hbm, dq_hbm, aq_sh)):
        pltpu.sync_copy(s_hbm.at[wid], sidx_v)
        pltpu.async_copy(x_hbm.at[sidx_v.at[0]], xg.at[0], gsems.at[0])
        pltpu.async_copy(ea_hbm.at[c0], eab.at[0], easems.at[0])
        pltpu.async_copy(d_hbm.at[wid, 0], didx_v.at[0], dsems.at[0])

        def _chunk(j, _):
            p = lax.rem(j, 2)
            q = 1 - p
            pltpu.make_async_copy(x_hbm.at[sidx_v.at[j]], xg.at[p],
                                  gsems.at[p]).wait()
            pltpu.make_async_copy(ea_hbm.at[c0 + j], eab.at[p],
                                  easems.at[p]).wait()
            pltpu.make_async_copy(d_hbm.at[wid, j], didx_v.at[p],
                                  dsems.at[p]).wait()

            @pl.when(j >= 1)
            def _():
                pltpu.make_async_copy(eab.at[q], a_sh.at[didx_v.at[q]],
                                      ssems.at[q]).wait()

            @pl.when(j + 1 < nc)
            def _():
                pltpu.async_copy(x_hbm.at[sidx_v.at[j + 1]], xg.at[q],
                                 gsems.at[q])
                pltpu.async_copy(ea_hbm.at[c0 + j + 1], eab.at[q],
                                 easems.at[q])
                pltpu.async_copy(d_hbm.at[wid, j + 1], didx_v.at[q],
                                 dsems.at[q])

            def _valu(r4, _):
                for rr in range(4):
                    for k in range(4):
                        sl = pl.ds(k * 16, 16)
                        a = xg[p, r4 * 4 + rr, sl]
                        b = eab[p, r4 * 4 + rr, sl]
                        eab[p, r4 * 4 + rr, sl] = jnp.maximum(a + b, 0.0)
                return 0

            lax.fori_loop(0, _C // 4, _valu, 0)
            pltpu.async_copy(eab.at[p], a_sh.at[didx_v.at[p]], ssems.at[p],
                             add=True)
            return 0

        lax.fori_loop(0, nc, _chunk, 0)
        pl_last = lax.rem(nc - 1, 2)
        pltpu.make_async_copy(eab.at[pl_last], a_sh.at[didx_v.at[pl_last]],
                              ssems.at[pl_last]).wait()

    plsc.subcore_barrier()

    @pl.when(sid < 15)
    def _():
        pltpu.sync_copy(ap_sh.at[pl.ds(r0, _RPT), :],
                        out_hbm.at[cid, 0, pl.ds(r0, _RPT), :])
        pltpu.sync_copy(aq_sh.at[pl.ds(r0, _RPT), :],
                        out_hbm.at[cid, 1, pl.ds(r0, _RPT), :])

    @pl.when(sid == 15)
    def _():
        pltpu.sync_copy(ap_sh.at[pl.ds(r0, _RPT_LAST), :],
                        out_hbm.at[cid, 0, pl.ds(r0, _RPT_LAST), :])
        pltpu.sync_copy(aq_sh.at[pl.ds(r0, _RPT_LAST), :],
                        out_hbm.at[cid, 1, pl.ds(r0, _RPT_LAST), :])


def _sc_layer(xp, xq, eap3, eaq3, sp3, dp3, sq3, dq3):
    mesh = plsc.VectorSubcoreMesh(core_axis_name="c", subcore_axis_name="s")
    f = pl.kernel(
        _sc_layer_body,
        out_type=jax.ShapeDtypeStruct((2, 2, N_NODES, HID), jnp.float32),
        mesh=mesh,
        scratch_types=[
            pltpu.VMEM_SHARED((N_NODES, HID), jnp.float32),
            pltpu.VMEM_SHARED((N_NODES, HID), jnp.float32),
            pltpu.VMEM((_CBASE + 1, _C), jnp.int32),
            pltpu.VMEM((2, _C), jnp.int32),
            pltpu.VMEM((2, _C, HID), jnp.float32),
            pltpu.VMEM((2, _C, HID), jnp.float32),
            pltpu.VMEM((8, HID), jnp.float32),
            pltpu.SemaphoreType.DMA((2,)),
            pltpu.SemaphoreType.DMA((2,)),
            pltpu.SemaphoreType.DMA((2,)),
            pltpu.SemaphoreType.DMA((2,)),
        ],
        compiler_params=pltpu.CompilerParams(use_tc_tiling_on_sc=False),
    )
    return f(xp, xq, eap3, eaq3, sp3, dp3, sq3, dq3)


def _proj_body(xp_ref, xq_ref, wp_ref, bp_ref, wq_ref, bq_ref, op_ref, oq_ref):
    op_ref[...] = jnp.dot(xp_ref[...], wp_ref[...],
                          preferred_element_type=jnp.float32) + bp_ref[...]
    oq_ref[...] = jnp.dot(xq_ref[...], wq_ref[...],
                          preferred_element_type=jnp.float32) + bq_ref[...]


def _proj_pair(xp, xq, pp, pq, nrows, din, blk):
    nblk = nrows // blk
    in_spec = pl.BlockSpec((blk, din), lambda i: (i, 0))
    out_spec = pl.BlockSpec((blk, HID), lambda i: (i, 0))
    full = lambda shape: pl.BlockSpec(shape, lambda i: (0, 0))
    out = pl.pallas_call(
        _proj_body,
        grid=(nblk,),
        in_specs=[in_spec, in_spec, full((din, HID)), full((1, HID)),
                  full((din, HID)), full((1, HID))],
        out_specs=[out_spec, out_spec],
        out_shape=[jax.ShapeDtypeStruct((nrows, HID), jnp.float32)] * 2,
    )(xp, xq, pp["w"], pp["b"].reshape(1, HID), pq["w"], pq["b"].reshape(1, HID))
    return out


def _mlp_body(x_ref, a_ref, w0_ref, b0_ref, w1_ref, b1_ref, sc_ref, bt_ref, o_ref):
    x = x_ref[...]
    y = x + a_ref[0, 0] + a_ref[1, 0]
    h = jnp.dot(y, w0_ref[...], preferred_element_type=jnp.float32) + b0_ref[...]
    h = jnp.maximum(h, 0.0)
    h = jnp.dot(h, w1_ref[...], preferred_element_type=jnp.float32) + b1_ref[...]
    h = sc_ref[...] * h / jnp.sqrt(1.0 + 1e-5) + bt_ref[...]
    o_ref[...] = x + jnp.maximum(h, 0.0)


def _mlp_layer(x, aggr4, enc, conv, norm):
    scale = norm["gamma"].reshape(1, HID)
    beta = norm["beta"].reshape(1, HID)
    b0 = conv["l0"]["b"].reshape(1, HID)
    b1 = conv["l1"]["b"].reshape(1, HID)
    nblk = N_NODES // _MLP_BLK
    row_spec = pl.BlockSpec((_MLP_BLK, HID), lambda i: (i, 0))
    a_spec = pl.BlockSpec((2, 1, _MLP_BLK, HID), lambda i: (0, enc, i, 0))
    full = lambda shape: pl.BlockSpec(shape, lambda i: (0, 0))
    return pl.pallas_call(
        _mlp_body,
        grid=(nblk,),
        in_specs=[row_spec, a_spec, full((HID, HID)), full((1, HID)),
                  full((HID, HID)), full((1, HID)), full((1, HID)), full((1, HID))],
        out_specs=row_spec,
        out_shape=jax.ShapeDtypeStruct((N_NODES, HID), jnp.float32),
    )(x, aggr4, conv["l0"]["w"], b0, conv["l1"]["w"], b1, scale, beta)


def _head_body(xp_ref, xq_ref, bp_ref, bq_ref, bpc_ref, bqc_ref, sh_ref,
               bf_ref, cf_ref, gf_ref, w_refs, mean_ref, logvar_ref,
               gmp_ref, gmq_ref):
    def dot(a, b):
        return jnp.dot(a, b, preferred_element_type=jnp.float32)

    def lin(a, i):
        return dot(a, w_refs[i][...]) + w_refs[i + 1][...]

    ones = jnp.ones((N_NODES, 1), jnp.float32)
    gs = []
    for x_ref, b_ref, bc_ref, gm_ref in ((xp_ref, bp_ref, bpc_ref, gmp_ref),
                                         (xq_ref, bq_ref, bqc_ref, gmq_ref)):
        iota = lax.broadcasted_iota(jnp.int32, (N_GRAPHS, N_NODES), 0)
        oh = (iota == b_ref[...]).astype(jnp.float32)
        ssum = dot(oh, x_ref[...])
        cnt = dot(oh, ones)
        gmean = ssum / jnp.maximum(cnt, 1.0)

        def _gmax(g, _):
            mask = bc_ref[...] == g
            masked = jnp.where(mask, x_ref[...], -3.4e38)
            gm_ref[pl.ds(g, 1), :] = jnp.max(masked, axis=0, keepdims=True)
            return 0

        lax.fori_loop(0, N_GRAPHS, _gmax, 0)
        gmax = jnp.where(cnt > 0, gm_ref[...], 0.0)
        gs.append(jnp.concatenate([gmean, gmax], axis=1))
    g_pre, g_post = gs
    pair = jnp.concatenate([g_pre, g_post], axis=1)
    gate_pre = jax.nn.sigmoid(lin(jax.nn.relu(lin(pair, 0)), 2))
    gate_post = jax.nn.sigmoid(lin(jax.nn.relu(lin(pair, 4)), 6))
    gp = gate_pre * g_pre
    gq = gate_post * g_post
    g_pair = jnp.concatenate([gp, gq, jnp.abs(gp - gq), gp * gq], axis=1)
    bz = jax.nn.relu(lin(jax.nn.relu(lin(bf_ref[...], 8)), 10))
    cz = jax.nn.relu(lin(jax.nn.relu(lin(cf_ref[...], 12)), 14))
    gz = jax.nn.relu(lin(jax.nn.relu(lin(gf_ref[...], 16)), 18))
    h = jnp.concatenate([g_pair, bz, cz, gz], axis=1)
    h = jax.nn.relu(lin(h, 20))
    h = jax.nn.relu(lin(h, 22))
    f_inf = jax.nn.sigmoid(lin(h, 24))
    theta = jax.nn.softplus(lin(h, 26)) + 1.0
    log_c = lin(h, 28)
    s = sh_ref[...]
    mean_ref[...] = f_inf * (s / (s + theta))
    logvar_ref[...] = log_c - jnp.log(s)


def _head(xp, xq, bp, bq, shots, backend_feat, circuit_feat,
          graph_attr_feat, params):
    def wb(p):
        return [p["w"], p["b"].reshape(1, -1)]

    ws = (wb(params["fusion_pre"]["l0"]) + wb(params["fusion_pre"]["l1"])
          + wb(params["fusion_post"]["l0"]) + wb(params["fusion_post"]["l1"])
          + wb(params["backend_enc"]["l0"]) + wb(params["backend_enc"]["l1"])
          + wb(params["circuit_enc"]["l0"]) + wb(params["circuit_enc"]["l1"])
          + wb(params["gattr_enc"]["l0"]) + wb(params["gattr_enc"]["l1"])
          + wb(params["backbone"]["l0"]) + wb(params["backbone"]["l1"])
          + wb(params["f_inf"]) + wb(params["theta"]) + wb(params["log_c"]))
    mean, logvar = pl.pallas_call(
        _head_body,
        out_shape=[jax.ShapeDtypeStruct((N_GRAPHS, 1), jnp.float32)] * 2,
        scratch_shapes=[pltpu.VMEM((N_GRAPHS, HID), jnp.float32)] * 2,
    )(xp, xq, bp.reshape(1, N_NODES), bq.reshape(1, N_NODES),
      bp.reshape(N_NODES, 1), bq.reshape(N_NODES, 1),
      shots.reshape(N_GRAPHS, 1), backend_feat, circuit_feat,
      graph_attr_feat, ws)
    return mean.reshape(N_GRAPHS), logvar.reshape(N_GRAPHS)


_WSTART = [w * _CBASE + min(w, _CEXTRA) for w in range(_NW)]


def _idx3d(idx):
    idx2 = jnp.pad(idx.reshape(_NCHUNK, _C), ((0, 8), (0, 0)))
    starts = jnp.asarray(_WSTART, jnp.int32)
    rows = starts[:, None] + jnp.arange(_CBASE + 1, dtype=jnp.int32)[None, :]
    return idx2[rows]


def kernel(pre_x, pre_edge_index, pre_edge_attr, pre_batch, post_x,
           post_edge_index, post_edge_attr, post_batch, shots, backend_feat,
           circuit_feat, graph_attr_feat, params):
    pp, pq = params["pre_enc"], params["post_enc"]
    xp, xq = _proj_pair(pre_x, post_x, pp["node_proj"], pq["node_proj"],
                        N_NODES, 128, 2000)
    eap, eaq = _proj_pair(pre_edge_attr, post_edge_attr, pp["edge_proj"],
                          pq["edge_proj"], N_EDGES, 16, 8000)
    eap3 = eap.reshape(_NCHUNK, _C, HID)
    eaq3 = eaq.reshape(_NCHUNK, _C, HID)
    sp2, dp2 = _idx3d(pre_edge_index[0]), _idx3d(pre_edge_index[1])
    sq2, dq2 = _idx3d(post_edge_index[0]), _idx3d(post_edge_index[1])
    for i in range(N_LAYERS):
        aggr4 = _sc_layer(xp, xq, eap3, eaq3, sp2, dp2, sq2, dq2)
        xp = _mlp_layer(xp, aggr4, 0, pp["convs"][i], pp["norms"][i])
        xq = _mlp_layer(xq, aggr4, 1, pq["convs"][i], pq["norms"][i])
    return _head(xp, xq, pre_batch, post_batch, shots, backend_feat,
                 circuit_feat, graph_attr_feat, params)


# speedup vs baseline: 3.3571x; 1.1156x over previous
import jax
import jax.numpy as jnp
from jax import lax
from jax.experimental import pallas as pl
from jax.experimental.pallas import tpu as pltpu
from jax.experimental.pallas import tpu_sc as plsc

HID = 64
N_NODES = 10000
N_EDGES = 320000
N_GRAPHS = 64
N_LAYERS = 4

_MLP_BLK = 2000

_C = 128
_NCHUNK = N_EDGES // _C
_NW = 32
_CBASE = _NCHUNK // _NW
_CEXTRA = _NCHUNK % _NW
_RPT = 632
_RPT_LAST = N_NODES - 15 * _RPT


def _sc_layer_body(xp_hbm, xq_hbm, eap_hbm, eaq_hbm, sp_hbm, dp_hbm, sq_hbm,
                   dq_hbm, z_hbm, out_hbm, ap_sh, aq_sh, sidx_v, didx_v, xg,
                   eab, gsems, easems, ssems, isems, dsems):
    cid = lax.axis_index("c")
    sid = lax.axis_index("s")
    wid = sid * 2 + cid
    c0 = wid * _CBASE + jnp.minimum(wid, _CEXTRA)
    nc = _CBASE + jnp.where(wid < _CEXTRA, 1, 0)
    r0 = sid * _RPT

    @pl.when(sid < 15)
    def _():
        pltpu.sync_copy(z_hbm, ap_sh.at[pl.ds(r0, _RPT), :])
        pltpu.sync_copy(z_hbm, aq_sh.at[pl.ds(r0, _RPT), :])

    @pl.when(sid == 15)
    def _():
        pltpu.sync_copy(z_hbm.at[pl.ds(0, _RPT_LAST), :],
                        ap_sh.at[pl.ds(r0, _RPT_LAST), :])
        pltpu.sync_copy(z_hbm.at[pl.ds(0, _RPT_LAST), :],
                        aq_sh.at[pl.ds(r0, _RPT_LAST), :])

    plsc.subcore_barrier()

    for x_hbm, ea_hbm, s_hbm, d_hbm, a_sh in (
            (xp_hbm, eap_hbm, sp_hbm, dp_hbm, ap_sh),
            (xq_hbm, eaq_hbm, sq_hbm, dq_hbm, aq_sh)):
        pltpu.async_copy(s_hbm.at[wid, 0], sidx_v.at[0], isems.at[0])
        pltpu.async_copy(d_hbm.at[wid, 0], didx_v.at[0], dsems.at[0])
        pltpu.async_copy(s_hbm.at[wid, 1], sidx_v.at[1], isems.at[1])
        pltpu.async_copy(d_hbm.at[wid, 1], didx_v.at[1], dsems.at[1])
        pltpu.make_async_copy(s_hbm.at[wid, 0], sidx_v.at[0],
                              isems.at[0]).wait()
        pltpu.async_copy(x_hbm.at[sidx_v.at[0]], xg.at[0], gsems.at[0])
        pltpu.async_copy(ea_hbm.at[c0], eab.at[0], easems.at[0])

        def _chunk(j, _):
            p = lax.rem(j, 3)
            i4 = lax.rem(j, 4)

            @pl.when(j + 2 < nc)
            def _():
                i4n = lax.rem(j + 2, 4)
                pltpu.async_copy(s_hbm.at[wid, j + 2], sidx_v.at[i4n],
                                 isems.at[i4n])
                pltpu.async_copy(d_hbm.at[wid, j + 2], didx_v.at[i4n],
                                 dsems.at[i4n])

            @pl.when(j + 1 < nc)
            def _():
                i4n = lax.rem(j + 1, 4)
                pn = lax.rem(j + 1, 3)
                pltpu.make_async_copy(s_hbm.at[wid, j + 1], sidx_v.at[i4n],
                                      isems.at[i4n]).wait()
                pltpu.async_copy(x_hbm.at[sidx_v.at[i4n]], xg.at[pn],
                                 gsems.at[pn])
                pltpu.async_copy(ea_hbm.at[c0 + j + 1], eab.at[pn],
                                 easems.at[pn])

            pltpu.make_async_copy(x_hbm.at[sidx_v.at[0]], xg.at[p],
                                  gsems.at[p]).wait()
            pltpu.make_async_copy(ea_hbm.at[c0 + j], eab.at[p],
                                  easems.at[p]).wait()

            def _valu(r8, _):
                for rr in range(8):
                    for k in range(4):
                        sl = pl.ds(k * 16, 16)
                        a = xg[p, r8 * 8 + rr, sl]
                        b = eab[p, r8 * 8 + rr, sl]
                        eab[p, r8 * 8 + rr, sl] = jnp.maximum(a + b, 0.0)
                return 0

            lax.fori_loop(0, _C // 8, _valu, 0)
            pltpu.make_async_copy(d_hbm.at[wid, j], didx_v.at[i4],
                                  dsems.at[i4]).wait()

            @pl.when(j >= 1)
            def _():
                pq = lax.rem(j - 1, 3)
                pltpu.make_async_copy(eab.at[pq], a_sh.at[didx_v.at[0]],
                                      ssems.at[pq]).wait()

            pltpu.async_copy(eab.at[p], a_sh.at[didx_v.at[i4]], ssems.at[p],
                             add=True)
            return 0

        lax.fori_loop(0, nc, _chunk, 0)
        pb = lax.rem(nc - 1, 3)
        pltpu.make_async_copy(eab.at[pb], a_sh.at[didx_v.at[0]],
                              ssems.at[pb]).wait()

    plsc.subcore_barrier()

    @pl.when(sid < 15)
    def _():
        pltpu.sync_copy(ap_sh.at[pl.ds(r0, _RPT), :],
                        out_hbm.at[cid, 0, pl.ds(r0, _RPT), :])
        pltpu.sync_copy(aq_sh.at[pl.ds(r0, _RPT), :],
                        out_hbm.at[cid, 1, pl.ds(r0, _RPT), :])

    @pl.when(sid == 15)
    def _():
        pltpu.sync_copy(ap_sh.at[pl.ds(r0, _RPT_LAST), :],
                        out_hbm.at[cid, 0, pl.ds(r0, _RPT_LAST), :])
        pltpu.sync_copy(aq_sh.at[pl.ds(r0, _RPT_LAST), :],
                        out_hbm.at[cid, 1, pl.ds(r0, _RPT_LAST), :])


def _sc_layer(xp, xq, eap3, eaq3, sp3, dp3, sq3, dq3):
    mesh = plsc.VectorSubcoreMesh(core_axis_name="c", subcore_axis_name="s")
    f = pl.kernel(
        _sc_layer_body,
        out_type=jax.ShapeDtypeStruct((2, 2, N_NODES, HID), jnp.float32),
        mesh=mesh,
        scratch_types=[
            pltpu.VMEM_SHARED((N_NODES, HID), jnp.float32),
            pltpu.VMEM_SHARED((N_NODES, HID), jnp.float32),
            pltpu.VMEM((4, _C), jnp.int32),
            pltpu.VMEM((4, _C), jnp.int32),
            pltpu.VMEM((3, _C, HID), jnp.float32),
            pltpu.VMEM((3, _C, HID), jnp.float32),
            pltpu.SemaphoreType.DMA((3,)),
            pltpu.SemaphoreType.DMA((3,)),
            pltpu.SemaphoreType.DMA((3,)),
            pltpu.SemaphoreType.DMA((4,)),
            pltpu.SemaphoreType.DMA((4,)),
        ],
        compiler_params=pltpu.CompilerParams(use_tc_tiling_on_sc=False),
    )
    zeros = jnp.zeros((_RPT, HID), jnp.float32)
    return f(xp, xq, eap3, eaq3, sp3, dp3, sq3, dq3, zeros)


def _proj_body(xp_ref, xq_ref, wp_ref, bp_ref, wq_ref, bq_ref, op_ref, oq_ref):
    op_ref[...] = jnp.dot(xp_ref[...], wp_ref[...],
                          preferred_element_type=jnp.float32) + bp_ref[...]
    oq_ref[...] = jnp.dot(xq_ref[...], wq_ref[...],
                          preferred_element_type=jnp.float32) + bq_ref[...]


def _proj_pair(xp, xq, pp, pq, nrows, din, blk):
    nblk = nrows // blk
    in_spec = pl.BlockSpec((blk, din), lambda i: (i, 0))
    out_spec = pl.BlockSpec((blk, HID), lambda i: (i, 0))
    full = lambda shape: pl.BlockSpec(shape, lambda i: (0, 0))
    out = pl.pallas_call(
        _proj_body,
        grid=(nblk,),
        in_specs=[in_spec, in_spec, full((din, HID)), full((1, HID)),
                  full((din, HID)), full((1, HID))],
        out_specs=[out_spec, out_spec],
        out_shape=[jax.ShapeDtypeStruct((nrows, HID), jnp.float32)] * 2,
    )(xp, xq, pp["w"], pp["b"].reshape(1, HID), pq["w"], pq["b"].reshape(1, HID))
    return out


def _mlp_body(x_ref, a_ref, w0_ref, b0_ref, w1_ref, b1_ref, sc_ref, bt_ref, o_ref):
    x = x_ref[...]
    y = x + a_ref[0, 0] + a_ref[1, 0]
    h = jnp.dot(y, w0_ref[...], preferred_element_type=jnp.float32) + b0_ref[...]
    h = jnp.maximum(h, 0.0)
    h = jnp.dot(h, w1_ref[...], preferred_element_type=jnp.float32) + b1_ref[...]
    h = sc_ref[...] * h / jnp.sqrt(1.0 + 1e-5) + bt_ref[...]
    o_ref[...] = x + jnp.maximum(h, 0.0)


def _mlp_layer(x, aggr4, enc, conv, norm):
    scale = norm["gamma"].reshape(1, HID)
    beta = norm["beta"].reshape(1, HID)
    b0 = conv["l0"]["b"].reshape(1, HID)
    b1 = conv["l1"]["b"].reshape(1, HID)
    nblk = N_NODES // _MLP_BLK
    row_spec = pl.BlockSpec((_MLP_BLK, HID), lambda i: (i, 0))
    a_spec = pl.BlockSpec((2, 1, _MLP_BLK, HID), lambda i: (0, enc, i, 0))
    full = lambda shape: pl.BlockSpec(shape, lambda i: (0, 0))
    return pl.pallas_call(
        _mlp_body,
        grid=(nblk,),
        in_specs=[row_spec, a_spec, full((HID, HID)), full((1, HID)),
                  full((HID, HID)), full((1, HID)), full((1, HID)), full((1, HID))],
        out_specs=row_spec,
        out_shape=jax.ShapeDtypeStruct((N_NODES, HID), jnp.float32),
    )(x, aggr4, conv["l0"]["w"], b0, conv["l1"]["w"], b1, scale, beta)


def _head_body(xp_ref, xq_ref, bp_ref, bq_ref, bpc_ref, bqc_ref, sh_ref,
               bf_ref, cf_ref, gf_ref, w_refs, mean_ref, logvar_ref,
               gmp_ref, gmq_ref):
    def dot(a, b):
        return jnp.dot(a, b, preferred_element_type=jnp.float32)

    def lin(a, i):
        return dot(a, w_refs[i][...]) + w_refs[i + 1][...]

    ones = jnp.ones((N_NODES, 1), jnp.float32)
    gs = []
    for x_ref, b_ref, bc_ref, gm_ref in ((xp_ref, bp_ref, bpc_ref, gmp_ref),
                                         (xq_ref, bq_ref, bqc_ref, gmq_ref)):
        iota = lax.broadcasted_iota(jnp.int32, (N_GRAPHS, N_NODES), 0)
        oh = (iota == b_ref[...]).astype(jnp.float32)
        ssum = dot(oh, x_ref[...])
        cnt = dot(oh, ones)
        gmean = ssum / jnp.maximum(cnt, 1.0)

        def _gmax(g, _):
            mask = bc_ref[...] == g
            masked = jnp.where(mask, x_ref[...], -3.4e38)
            gm_ref[pl.ds(g, 1), :] = jnp.max(masked, axis=0, keepdims=True)
            return 0

        lax.fori_loop(0, N_GRAPHS, _gmax, 0)
        gmax = jnp.where(cnt > 0, gm_ref[...], 0.0)
        gs.append(jnp.concatenate([gmean, gmax], axis=1))
    g_pre, g_post = gs
    pair = jnp.concatenate([g_pre, g_post], axis=1)
    gate_pre = jax.nn.sigmoid(lin(jax.nn.relu(lin(pair, 0)), 2))
    gate_post = jax.nn.sigmoid(lin(jax.nn.relu(lin(pair, 4)), 6))
    gp = gate_pre * g_pre
    gq = gate_post * g_post
    g_pair = jnp.concatenate([gp, gq, jnp.abs(gp - gq), gp * gq], axis=1)
    bz = jax.nn.relu(lin(jax.nn.relu(lin(bf_ref[...], 8)), 10))
    cz = jax.nn.relu(lin(jax.nn.relu(lin(cf_ref[...], 12)), 14))
    gz = jax.nn.relu(lin(jax.nn.relu(lin(gf_ref[...], 16)), 18))
    h = jnp.concatenate([g_pair, bz, cz, gz], axis=1)
    h = jax.nn.relu(lin(h, 20))
    h = jax.nn.relu(lin(h, 22))
    f_inf = jax.nn.sigmoid(lin(h, 24))
    theta = jax.nn.softplus(lin(h, 26)) + 1.0
    log_c = lin(h, 28)
    s = sh_ref[...]
    mean_ref[...] = f_inf * (s / (s + theta))
    logvar_ref[...] = log_c - jnp.log(s)


def _head(xp, xq, bp, bq, shots, backend_feat, circuit_feat,
          graph_attr_feat, params):
    def wb(p):
        return [p["w"], p["b"].reshape(1, -1)]

    ws = (wb(params["fusion_pre"]["l0"]) + wb(params["fusion_pre"]["l1"])
          + wb(params["fusion_post"]["l0"]) + wb(params["fusion_post"]["l1"])
          + wb(params["backend_enc"]["l0"]) + wb(params["backend_enc"]["l1"])
          + wb(params["circuit_enc"]["l0"]) + wb(params["circuit_enc"]["l1"])
          + wb(params["gattr_enc"]["l0"]) + wb(params["gattr_enc"]["l1"])
          + wb(params["backbone"]["l0"]) + wb(params["backbone"]["l1"])
          + wb(params["f_inf"]) + wb(params["theta"]) + wb(params["log_c"]))
    mean, logvar = pl.pallas_call(
        _head_body,
        out_shape=[jax.ShapeDtypeStruct((N_GRAPHS, 1), jnp.float32)] * 2,
        scratch_shapes=[pltpu.VMEM((N_GRAPHS, HID), jnp.float32)] * 2,
    )(xp, xq, bp.reshape(1, N_NODES), bq.reshape(1, N_NODES),
      bp.reshape(N_NODES, 1), bq.reshape(N_NODES, 1),
      shots.reshape(N_GRAPHS, 1), backend_feat, circuit_feat,
      graph_attr_feat, ws)
    return mean.reshape(N_GRAPHS), logvar.reshape(N_GRAPHS)


_WSTART = [w * _CBASE + min(w, _CEXTRA) for w in range(_NW)]


def _idx3d(idx):
    idx2 = jnp.pad(idx.reshape(_NCHUNK, _C), ((0, 8), (0, 0)))
    starts = jnp.asarray(_WSTART, jnp.int32)
    rows = starts[:, None] + jnp.arange(_CBASE + 1, dtype=jnp.int32)[None, :]
    return idx2[rows]


def kernel(pre_x, pre_edge_index, pre_edge_attr, pre_batch, post_x,
           post_edge_index, post_edge_attr, post_batch, shots, backend_feat,
           circuit_feat, graph_attr_feat, params):
    pp, pq = params["pre_enc"], params["post_enc"]
    xp, xq = _proj_pair(pre_x, post_x, pp["node_proj"], pq["node_proj"],
                        N_NODES, 128, 2000)
    eap, eaq = _proj_pair(pre_edge_attr, post_edge_attr, pp["edge_proj"],
                          pq["edge_proj"], N_EDGES, 16, 8000)
    eap3 = eap.reshape(_NCHUNK, _C, HID)
    eaq3 = eaq.reshape(_NCHUNK, _C, HID)
    sp2, dp2 = _idx3d(pre_edge_index[0]), _idx3d(pre_edge_index[1])
    sq2, dq2 = _idx3d(post_edge_index[0]), _idx3d(post_edge_index[1])
    for i in range(N_LAYERS):
        aggr4 = _sc_layer(xp, xq, eap3, eaq3, sp2, dp2, sq2, dq2)
        xp = _mlp_layer(xp, aggr4, 0, pp["convs"][i], pp["norms"][i])
        xq = _mlp_layer(xq, aggr4, 1, pq["convs"][i], pq["norms"][i])
    return _head(xp, xq, pre_batch, post_batch, shots, backend_feat,
                 circuit_feat, graph_attr_feat, params)
